# Initial kernel scaffold; baseline (speedup 1.0000x reference)
import jax
import jax.numpy as jnp
from jax.experimental import pallas as pl


def kernel(task_id, target_ratio_tensor, task_embed_table, ratio_proj_w, ratio_proj_b):
    raise NotImplementedError("write your pallas kernel here")


# speedup vs baseline: 2.1271x; 2.1271x over previous
import functools
import jax
import jax.numpy as jnp
from jax import lax
from jax.experimental import pallas as pl
from jax.experimental.pallas import tpu as pltpu
from jax.experimental.pallas import tpu_sc as plsc

_D = 128
_L = 16
_NCHUNK = _D // _L
_B = 16384
_NW = 32
_PER = _B // _NW


def _sc_body(tid_hbm, ratio_hbm, table_hbm, w_hbm, b_hbm, out_hbm,
             tid_v, ratio_v, tab_v, w_v, b_v, out_v):
    wid = lax.axis_index("s") * 2 + lax.axis_index("c")
    base = wid * _PER

    pltpu.sync_copy(tid_hbm.at[pl.ds(base, _PER)], tid_v)
    pltpu.sync_copy(ratio_hbm.at[pl.ds(base, _PER)], ratio_v)
    pltpu.sync_copy(table_hbm, tab_v)
    pltpu.sync_copy(w_hbm, w_v)
    pltpu.sync_copy(b_hbm, b_v)

    row0p = []
    diff = []
    wk = []
    for k in range(_NCHUNK):
        sl = pl.ds(k * _L, _L)
        r0 = tab_v[0, sl]
        r1 = tab_v[1, sl]
        row0p.append(r0 + b_v[sl])
        diff.append(r1 - r0)
        wk.append(w_v[sl])

    def group(g, _):
        base16 = g * _L
        tidf = tid_v[pl.ds(base16, _L)].astype(jnp.float32)
        rv = ratio_v[pl.ds(base16, _L)]
        for j in range(_L):
            tf = tidf[j]
            r = rv[j]
            i = base16 + j
            for k in range(_NCHUNK):
                out_v[i, pl.ds(k * _L, _L)] = row0p[k] + tf * diff[k] + r * wk[k]
        return _

    lax.fori_loop(0, _PER // _L, group, None)
    pltpu.sync_copy(out_v, out_hbm.at[pl.ds(base, _PER)])


@jax.jit
def _run(tid, ratio_flat, table, w_flat, bias):
    mesh = plsc.VectorSubcoreMesh(core_axis_name="c", subcore_axis_name="s")
    fn = pl.kernel(
        _sc_body,
        out_type=jax.ShapeDtypeStruct((_B, _D), jnp.float32),
        mesh=mesh,
        scratch_types=[
            pltpu.VMEM((_PER,), jnp.int32),
            pltpu.VMEM((_PER,), jnp.float32),
            pltpu.VMEM((2, _D), jnp.float32),
            pltpu.VMEM((_D,), jnp.float32),
            pltpu.VMEM((_D,), jnp.float32),
            pltpu.VMEM((_PER, _D), jnp.float32),
        ],
    )
    return fn(tid, ratio_flat, table, w_flat, bias)


def kernel(task_id, target_ratio_tensor, task_embed_table, ratio_proj_w, ratio_proj_b):
    tid = task_id.astype(jnp.int32)
    ratio_flat = target_ratio_tensor.reshape(_B)
    w_flat = ratio_proj_w.reshape(_D)
    out = _run(tid, ratio_flat, task_embed_table, w_flat, ratio_proj_b)
    return out[:, None, :]
